# speedup vs baseline: 1.6259x; 1.6259x over previous
import jax
import jax.numpy as jnp
from jax.experimental import pallas as pl


def _mm_kernel(x_ref, w_ref, o_ref):
    o_ref[...] = jnp.dot(x_ref[...], w_ref[...],
                         preferred_element_type=jnp.float32)


def _matmul(x, w):
    return pl.pallas_call(
        _mm_kernel,
        out_shape=jax.ShapeDtypeStruct((x.shape[0], w.shape[1]), jnp.float32),
    )(x, w)


def _gat_conv(x, src, dst, W, a_src, a_dst, b):
    n = x.shape[0]
    h = _matmul(x, W)
    alpha_src = (h * a_src).sum(axis=-1)
    alpha_dst = (h * a_dst).sum(axis=-1)
    e = alpha_src[src] + alpha_dst[dst]
    e = jax.nn.leaky_relu(e, negative_slope=0.2)
    m = jax.nn.leaky_relu(jnp.max(alpha_src) + jnp.max(alpha_dst), 0.2)
    w_e = jnp.exp(e - m)
    denom = jax.ops.segment_sum(w_e, dst, num_segments=n)
    acc = jax.ops.segment_sum(h[src] * w_e[:, None], dst, num_segments=n)
    return acc / denom[:, None] + b


def kernel(x, edges_index, W1, a_src1, a_dst1, b1, W2, a_src2, a_dst2, b2):
    n = x.shape[0]
    loop = jnp.arange(n, dtype=edges_index.dtype)
    src = jnp.concatenate([edges_index[0], loop])
    dst = jnp.concatenate([edges_index[1], loop])
    h = _gat_conv(x, src, dst, W1, a_src1, a_dst1, b1)
    h = jax.nn.relu(h)
    h = _gat_conv(h, src, dst, W2, a_src2, a_dst2, b2)
    return jax.nn.log_softmax(h, axis=1)


# speedup vs baseline: 13.5401x; 8.3277x over previous
import dataclasses
import functools

import jax
import jax.numpy as jnp
from jax import lax
from jax.experimental import pallas as pl
from jax.experimental.pallas import tpu as pltpu
from jax.experimental.pallas import tpu_sc as plsc

N = 10000
NP = 10240
E = 320000
NC = 2
NS = 16
NW = NC * NS
CH = 128
CPT = 81
E_PAD = NW * CH * CPT
ROWS_PER_TEC = NP // NS
DEN_R = NP // 128
NEG = -1e30


def _f32(shape):
    return jax.ShapeDtypeStruct(shape, jnp.float32)


def _tc1_body(x_ref, w_ref, as_ref, ad_ref,
              h_ref, asrc_ref, adst_ref, m_ref):
    h = jnp.dot(x_ref[...], w_ref[...], preferred_element_type=jnp.float32)
    h_ref[:N, :] = h
    h_ref[N:, :] = jnp.zeros((NP - N, h.shape[1]), jnp.float32)
    asrc = jnp.sum(h * as_ref[...], axis=1)
    adst = jnp.sum(h * ad_ref[...], axis=1)
    asrc_ref[0:1, :N] = asrc[None, :]
    asrc_ref[0:1, N:] = jnp.full((1, NP - N), NEG, jnp.float32)
    adst_ref[0:1, :N] = adst[None, :]
    adst_ref[0:1, N:] = jnp.full((1, NP - N), NEG, jnp.float32)
    mm = jnp.max(asrc) + jnp.max(adst)
    m = jnp.maximum(mm, 0.2 * mm)
    m_ref[0:1, :] = jnp.full((1, 16), m, jnp.float32)


def _tc1(x, W1, a_src1, a_dst1, d_out):
    return pl.pallas_call(
        _tc1_body,
        out_shape=(_f32((NP, d_out)), _f32((1, NP)), _f32((1, NP)),
                   _f32((1, 16))),
    )(x, W1, a_src1, a_dst1)


def _tc2_body(acc_ref, den_ref, b_ref, w_ref, as_ref, ad_ref,
              h_ref, asrc_ref, adst_ref, m_ref):
    den = jnp.sum(den_ref[...], axis=0).reshape(NP)
    acc = acc_ref[0] + acc_ref[1]
    h1 = acc[:N, :] / den[:N, None] + b_ref[...]
    h1 = jnp.maximum(h1, 0.0)
    h2 = jnp.dot(h1, w_ref[...], preferred_element_type=jnp.float32)
    d2 = h2.shape[1]
    d2p = h_ref.shape[1]
    h_ref[:N, :d2] = h2
    h_ref[:N, d2:] = jnp.zeros((N, d2p - d2), jnp.float32)
    h_ref[N:, :] = jnp.zeros((NP - N, d2p), jnp.float32)
    asrc = jnp.sum(h2 * as_ref[...], axis=1)
    adst = jnp.sum(h2 * ad_ref[...], axis=1)
    asrc_ref[0:1, :N] = asrc[None, :]
    asrc_ref[0:1, N:] = jnp.full((1, NP - N), NEG, jnp.float32)
    adst_ref[0:1, :N] = adst[None, :]
    adst_ref[0:1, N:] = jnp.full((1, NP - N), NEG, jnp.float32)
    mm = jnp.max(asrc) + jnp.max(adst)
    m = jnp.maximum(mm, 0.2 * mm)
    m_ref[0:1, :] = jnp.full((1, 16), m, jnp.float32)


def _tc2(acc, den, b1, W2, a_src2, a_dst2, d2p):
    return pl.pallas_call(
        _tc2_body,
        out_shape=(_f32((NP, d2p)), _f32((1, NP)), _f32((1, NP)),
                   _f32((1, 16))),
    )(acc, den, b1, W2, a_src2, a_dst2)


def _tc3_body(acc_ref, den_ref, b_ref, out_ref):
    den = jnp.sum(den_ref[...], axis=0).reshape(NP)
    d_out = out_ref.shape[1]
    g = (acc_ref[0] + acc_ref[1])[:N, :d_out]
    v = g / den[:N, None] + b_ref[...]
    v = v - jnp.max(v, axis=1, keepdims=True)
    out_ref[...] = v - jnp.log(jnp.sum(jnp.exp(v), axis=1, keepdims=True))


def _tc3(acc, den, b2, d_out):
    return pl.pallas_call(
        _tc3_body,
        out_shape=_f32((N, d_out)),
    )(acc, den, b2)


def _sc_compiler_params():
    cp = pltpu.CompilerParams()
    fields = pltpu.CompilerParams.__dataclass_fields__
    if "needs_layout_passes" in fields:
        cp = dataclasses.replace(cp, needs_layout_passes=False)
    if "use_tc_tiling_on_sc" in fields:
        cp = dataclasses.replace(cp, use_tc_tiling_on_sc=False)
    return cp


def _sc_edges(h_pad, asrc, adst, m16, src_pad, dst_pad, zer, d):
    mesh = plsc.VectorSubcoreMesh(core_axis_name="c", subcore_axis_name="s")

    @functools.partial(
        pl.kernel,
        out_type=(_f32((NC, NP, d)), _f32((NW, DEN_R, 128))),
        mesh=mesh,
        compiler_params=_sc_compiler_params(),
        scratch_types=[
            pltpu.VMEM_SHARED((NP, d), jnp.float32),
            pltpu.VMEM((CH,), jnp.int32),
            pltpu.VMEM((CH,), jnp.int32),
            pltpu.VMEM((CH, d), jnp.float32),
            pltpu.VMEM((NP,), jnp.float32),
            pltpu.VMEM((NP,), jnp.float32),
            pltpu.VMEM((DEN_R, 128), jnp.float32),
            pltpu.VMEM((16,), jnp.float32),
        ],
    )
    def k(h_hbm, asrc_hbm, adst_hbm, m_hbm, src_hbm, dst_hbm, zer_hbm,
          acc_hbm, den_hbm,
          acc_sh, src_v, dst_v, rows_v, asrc_v, adst_v, den_v, m_v):
        cid = lax.axis_index("c")
        sid = lax.axis_index("s")
        wid = cid * NS + sid
        pltpu.sync_copy(asrc_hbm, asrc_v)
        pltpu.sync_copy(adst_hbm, adst_v)
        pltpu.sync_copy(m_hbm, m_v)
        row0 = sid * ROWS_PER_TEC
        pltpu.sync_copy(zer_hbm.at[pl.ds(row0, ROWS_PER_TEC)],
                        acc_sh.at[pl.ds(row0, ROWS_PER_TEC)])
        z16 = jnp.zeros((16,), jnp.float32)

        @pl.loop(0, DEN_R)
        def _(r):
            @pl.loop(0, 128, step=16)
            def _(c):
                den_v[r, pl.ds(c, 16)] = z16

        m_reg = m_v[...]
        plsc.subcore_barrier()

        @pl.loop(0, CPT)
        def _(ci):
            base = wid * (CPT * CH) + ci * CH
            pltpu.sync_copy(src_hbm.at[pl.ds(base, CH)], src_v)
            pltpu.sync_copy(dst_hbm.at[pl.ds(base, CH)], dst_v)
            pltpu.sync_copy(h_hbm.at[src_v], rows_v)

            @pl.loop(0, CH, step=16)
            def _(g):
                src16 = src_v[pl.ds(g, 16)]
                dst16 = dst_v[pl.ds(g, 16)]
                s = plsc.load_gather(asrc_v, [src16])
                dd = plsc.load_gather(adst_v, [dst16])
                e = s + dd
                e = jnp.maximum(e, 0.2 * e)
                w16 = jnp.exp(e - m_reg)
                plsc.addupdate_scatter(
                    den_v,
                    [lax.shift_right_logical(dst16, 7),
                     jnp.bitwise_and(dst16, 127)], w16)
                row16 = lax.iota(jnp.int32, 16) + g
                for c in range(d):
                    col16 = jnp.full((16,), c, jnp.int32)
                    v = plsc.load_gather(rows_v, [row16, col16])
                    plsc.store_scatter(rows_v, [row16, col16], v * w16)

            pltpu.sync_copy(rows_v, acc_sh.at[dst_v], add=True)

        plsc.subcore_barrier()
        pltpu.sync_copy(acc_sh.at[pl.ds(row0, ROWS_PER_TEC)],
                        acc_hbm.at[cid].at[pl.ds(row0, ROWS_PER_TEC)])
        pltpu.sync_copy(den_v, den_hbm.at[wid])

    return k(h_pad, asrc, adst, m16, src_pad, dst_pad, zer)


def kernel(x, edges_index, W1, a_src1, a_dst1, b1, W2, a_src2, a_dst2, b2):
    d1 = W1.shape[1]
    d2 = W2.shape[1]
    d2p = 48
    loop = jnp.arange(N, dtype=edges_index.dtype)
    pad = jnp.full((E_PAD - E - N,), N, dtype=edges_index.dtype)
    src = jnp.concatenate([edges_index[0], loop, pad])
    dst = jnp.concatenate([edges_index[1], loop, pad])

    zer1 = jnp.zeros((NP, d1), jnp.float32)
    zer2 = jnp.zeros((NP, d2p), jnp.float32)

    h1, asrc1, adst1, m1 = _tc1(x, W1, a_src1, a_dst1, d1)
    acc1, den1 = _sc_edges(h1, asrc1.reshape(NP), adst1.reshape(NP),
                           m1.reshape(16), src, dst, zer1, d1)
    h2, asrc2, adst2, m2 = _tc2(acc1, den1, b1, W2, a_src2, a_dst2, d2p)
    acc2, den2 = _sc_edges(h2, asrc2.reshape(NP), adst2.reshape(NP),
                           m2.reshape(16), src, dst, zer2, d2p)
    return _tc3(acc2, den2, b2, d2)


# speedup vs baseline: 17.1222x; 1.2646x over previous
import dataclasses
import functools

import jax
import jax.numpy as jnp
from jax import lax
from jax.experimental import pallas as pl
from jax.experimental.pallas import tpu as pltpu
from jax.experimental.pallas import tpu_sc as plsc

N = 10000
NP = 10240
E = 320000
NC = 2
NS = 16
NW = NC * NS
CH = 128
CPT = 84
E_PAD = NW * CH * CPT
ROWS_PER_TEC = NP // NS
NEG = -1e30
D1 = 64
D1E = 80
D2 = 40
D2E = 48


def _f32(shape):
    return jax.ShapeDtypeStruct(shape, jnp.float32)


def _tc1_body(x_ref, w_ref, as_ref, ad_ref,
              h_ref, asrc_ref, adst_ref, m_ref):
    h = jnp.dot(x_ref[...], w_ref[...], preferred_element_type=jnp.float32)
    h_ref[:N, :D1] = h
    h_ref[N:, :D1] = jnp.zeros((NP - N, D1), jnp.float32)
    h_ref[:, D1:D1 + 1] = jnp.ones((NP, 1), jnp.float32)
    h_ref[:, D1 + 1:] = jnp.zeros((NP, D1E - D1 - 1), jnp.float32)
    asrc = jnp.sum(h * as_ref[...], axis=1)
    adst = jnp.sum(h * ad_ref[...], axis=1)
    asrc_ref[0:1, :N] = asrc[None, :]
    asrc_ref[0:1, N:] = jnp.full((1, NP - N), NEG, jnp.float32)
    adst_ref[0:1, :N] = adst[None, :]
    adst_ref[0:1, N:] = jnp.full((1, NP - N), NEG, jnp.float32)
    mm = jnp.max(asrc) + jnp.max(adst)
    m = jnp.maximum(mm, 0.2 * mm)
    m_ref[0:1, :] = jnp.full((1, 16), m, jnp.float32)


def _tc1(x, W1, a_src1, a_dst1):
    return pl.pallas_call(
        _tc1_body,
        out_shape=(_f32((NP, D1E)), _f32((1, NP)), _f32((1, NP)),
                   _f32((1, 16))),
    )(x, W1, a_src1, a_dst1)


def _tc2_body(acc_ref, b_ref, w_ref, as_ref, ad_ref,
              h_ref, asrc_ref, adst_ref, m_ref):
    acc = acc_ref[0] + acc_ref[1]
    h1 = acc[:N, :D1] / acc[:N, D1:D1 + 1] + b_ref[...]
    h1 = jnp.maximum(h1, 0.0)
    h2 = jnp.dot(h1, w_ref[...], preferred_element_type=jnp.float32)
    h_ref[:N, :D2] = h2
    h_ref[N:, :D2] = jnp.zeros((NP - N, D2), jnp.float32)
    h_ref[:, D2:D2 + 1] = jnp.ones((NP, 1), jnp.float32)
    h_ref[:, D2 + 1:] = jnp.zeros((NP, D2E - D2 - 1), jnp.float32)
    asrc = jnp.sum(h2 * as_ref[...], axis=1)
    adst = jnp.sum(h2 * ad_ref[...], axis=1)
    asrc_ref[0:1, :N] = asrc[None, :]
    asrc_ref[0:1, N:] = jnp.full((1, NP - N), NEG, jnp.float32)
    adst_ref[0:1, :N] = adst[None, :]
    adst_ref[0:1, N:] = jnp.full((1, NP - N), NEG, jnp.float32)
    mm = jnp.max(asrc) + jnp.max(adst)
    m = jnp.maximum(mm, 0.2 * mm)
    m_ref[0:1, :] = jnp.full((1, 16), m, jnp.float32)


def _tc2(acc, b1, W2, a_src2, a_dst2):
    return pl.pallas_call(
        _tc2_body,
        out_shape=(_f32((NP, D2E)), _f32((1, NP)), _f32((1, NP)),
                   _f32((1, 16))),
    )(acc, b1, W2, a_src2, a_dst2)


def _tc3_body(acc_ref, b_ref, out_ref):
    acc = acc_ref[0] + acc_ref[1]
    v = acc[:N, :D2] / acc[:N, D2:D2 + 1] + b_ref[...]
    v = v - jnp.max(v, axis=1, keepdims=True)
    out_ref[...] = v - jnp.log(jnp.sum(jnp.exp(v), axis=1, keepdims=True))


def _tc3(acc, b2):
    return pl.pallas_call(
        _tc3_body,
        out_shape=_f32((N, D2)),
    )(acc, b2)


def _sc_compiler_params():
    cp = pltpu.CompilerParams()
    fields = pltpu.CompilerParams.__dataclass_fields__
    if "needs_layout_passes" in fields:
        cp = dataclasses.replace(cp, needs_layout_passes=False)
    if "use_tc_tiling_on_sc" in fields:
        cp = dataclasses.replace(cp, use_tc_tiling_on_sc=False)
    return cp


def _sc_edges(h_pad, asrc, adst, m16, src_pad, dst_pad, zer, de, ds):
    mesh = plsc.VectorSubcoreMesh(core_axis_name="c", subcore_axis_name="s")

    @functools.partial(
        pl.kernel,
        out_type=_f32((NC, NP, de)),
        mesh=mesh,
        compiler_params=_sc_compiler_params(),
        scratch_types=[
            pltpu.VMEM_SHARED((NP, de), jnp.float32),
            pltpu.VMEM((CPT + 2, CH), jnp.int32),
            pltpu.VMEM((CPT, CH), jnp.int32),
            pltpu.VMEM((CH, de), jnp.float32),
            pltpu.VMEM((CH, de), jnp.float32),
            pltpu.VMEM((NP,), jnp.float32),
            pltpu.VMEM((NP,), jnp.float32),
            pltpu.VMEM((16,), jnp.float32),
            pltpu.SemaphoreType.DMA,
            pltpu.SemaphoreType.DMA,
        ],
    )
    def k(h_hbm, asrc_hbm, adst_hbm, m_hbm, src_hbm, dst_hbm, zer_hbm,
          acc_hbm,
          acc_sh, src_all, dst_all, rows0, rows1,
          asrc_v, adst_v, m_v, sem0, sem1):
        cid = lax.axis_index("c")
        sid = lax.axis_index("s")
        wid = cid * NS + sid
        pltpu.sync_copy(asrc_hbm, asrc_v)
        pltpu.sync_copy(adst_hbm, adst_v)
        pltpu.sync_copy(m_hbm, m_v)
        pltpu.sync_copy(src_hbm.at[wid], src_all)
        pltpu.sync_copy(dst_hbm.at[wid], dst_all)
        row0 = sid * ROWS_PER_TEC
        pltpu.sync_copy(zer_hbm.at[pl.ds(row0, ROWS_PER_TEC)],
                        acc_sh.at[pl.ds(row0, ROWS_PER_TEC)])
        m_reg = m_v[...]
        plsc.subcore_barrier()

        pltpu.async_copy(h_hbm.at[src_all.at[0]], rows0, sem0)
        pltpu.async_copy(h_hbm.at[src_all.at[1]], rows1, sem1)

        @pl.loop(0, CPT, step=2)
        def _(ci0):
            for b, (rows_v, sem) in enumerate(((rows0, sem0), (rows1, sem1))):
                ci = ci0 + b
                pltpu.make_async_copy(h_hbm.at[src_all.at[ci]], rows_v,
                                      sem).wait()

                @pl.loop(0, CH, step=16)
                def _(g):
                    src16 = src_all[ci, pl.ds(g, 16)]
                    dst16 = dst_all[ci, pl.ds(g, 16)]
                    s = plsc.load_gather(asrc_v, [src16])
                    dd = plsc.load_gather(adst_v, [dst16])
                    e = s + dd
                    e = jnp.maximum(e, 0.2 * e)
                    w16 = jnp.exp(e - m_reg)
                    row16 = lax.iota(jnp.int32, 16) + g
                    for c in range(ds):
                        col16 = jnp.full((16,), c, jnp.int32)
                        v = plsc.load_gather(rows_v, [row16, col16])
                        plsc.store_scatter(rows_v, [row16, col16], v * w16)

                pltpu.sync_copy(rows_v, acc_sh.at[dst_all.at[ci]], add=True)
                pltpu.async_copy(h_hbm.at[src_all.at[ci + 2]], rows_v, sem)

        pltpu.make_async_copy(h_hbm.at[src_all.at[CPT]], rows0, sem0).wait()
        pltpu.make_async_copy(h_hbm.at[src_all.at[CPT + 1]], rows1,
                              sem1).wait()
        plsc.subcore_barrier()
        pltpu.sync_copy(acc_sh.at[pl.ds(row0, ROWS_PER_TEC)],
                        acc_hbm.at[cid].at[pl.ds(row0, ROWS_PER_TEC)])

    return k(h_pad, asrc, adst, m16, src_pad, dst_pad, zer)


def kernel(x, edges_index, W1, a_src1, a_dst1, b1, W2, a_src2, a_dst2, b2):
    loop = jnp.arange(N, dtype=edges_index.dtype)
    pad = jnp.full((E_PAD - E - N,), N, dtype=edges_index.dtype)
    tail = jnp.full((NW, 2, CH), N, dtype=edges_index.dtype)
    src = jnp.concatenate([edges_index[0], loop, pad]).reshape(NW, CPT, CH)
    src = jnp.concatenate([src, tail], axis=1)
    dst = jnp.concatenate([edges_index[1], loop, pad]).reshape(NW, CPT, CH)

    zer1 = jnp.zeros((NP, D1E), jnp.float32)
    zer2 = jnp.zeros((NP, D2E), jnp.float32)

    h1, asrc1, adst1, m1 = _tc1(x, W1, a_src1, a_dst1)
    acc1 = _sc_edges(h1, asrc1.reshape(NP), adst1.reshape(NP),
                     m1.reshape(16), src, dst, zer1, D1E, D1 + 1)
    h2, asrc2, adst2, m2 = _tc2(acc1, b1, W2, a_src2, a_dst2)
    acc2 = _sc_edges(h2, asrc2.reshape(NP), adst2.reshape(NP),
                     m2.reshape(16), src, dst, zer2, D2E, D2 + 1)
    return _tc3(acc2, b2)


# speedup vs baseline: 18.3016x; 1.0689x over previous
import dataclasses
import functools

import jax
import jax.numpy as jnp
from jax import lax
from jax.experimental import pallas as pl
from jax.experimental.pallas import tpu as pltpu
from jax.experimental.pallas import tpu_sc as plsc

N = 10000
NP = 10240
E = 320000
NC = 2
NS = 16
NW = NC * NS
CH = 128
CPT = 84
E_PAD = NW * CH * CPT
ROWS_PER_TEC = NP // NS
NEG = -1e30
D1 = 64
D1E = 72
D2 = 40
D2E = 48


def _f32(shape):
    return jax.ShapeDtypeStruct(shape, jnp.float32)


def _fill_row(h_ref, h, asrc_col, d, de):
    h_ref[:N, :d] = h
    h_ref[N:, :d] = jnp.zeros((NP - N, d), jnp.float32)
    h_ref[:, d:d + 1] = jnp.ones((NP, 1), jnp.float32)
    h_ref[:N, d + 1:d + 2] = asrc_col
    h_ref[N:, d + 1:d + 2] = jnp.full((NP - N, 1), NEG, jnp.float32)
    h_ref[:, d + 2:] = jnp.zeros((NP, de - d - 2), jnp.float32)


def _tc1_body(x_ref, w_ref, as_ref, ad_ref, h_ref, adst_ref, m_ref):
    h = jnp.dot(x_ref[...], w_ref[...], preferred_element_type=jnp.float32)
    asrc_col = jnp.dot(h, as_ref[...][:, None],
                       preferred_element_type=jnp.float32)
    _fill_row(h_ref, h, asrc_col, D1, D1E)
    adst = jnp.sum(h * ad_ref[...], axis=1)
    adst_ref[0:1, :N] = adst[None, :]
    adst_ref[0:1, N:] = jnp.full((1, NP - N), NEG, jnp.float32)
    mm = jnp.max(asrc_col) + jnp.max(adst)
    m = jnp.maximum(mm, 0.2 * mm)
    m_ref[0:1, :] = jnp.full((1, 16), m, jnp.float32)


def _tc1(x, W1, a_src1, a_dst1):
    return pl.pallas_call(
        _tc1_body,
        out_shape=(_f32((NP, D1E)), _f32((1, NP)), _f32((1, 16))),
    )(x, W1, a_src1, a_dst1)


def _tc2_body(acc_ref, b_ref, w_ref, as_ref, ad_ref,
              h_ref, adst_ref, m_ref):
    acc = acc_ref[0] + acc_ref[1]
    h1 = acc[:N, :D1] / acc[:N, D1:D1 + 1] + b_ref[...]
    h1 = jnp.maximum(h1, 0.0)
    h2 = jnp.dot(h1, w_ref[...], preferred_element_type=jnp.float32)
    asrc_col = jnp.dot(h2, as_ref[...][:, None],
                       preferred_element_type=jnp.float32)
    _fill_row(h_ref, h2, asrc_col, D2, D2E)
    adst = jnp.sum(h2 * ad_ref[...], axis=1)
    adst_ref[0:1, :N] = adst[None, :]
    adst_ref[0:1, N:] = jnp.full((1, NP - N), NEG, jnp.float32)
    mm = jnp.max(asrc_col) + jnp.max(adst)
    m = jnp.maximum(mm, 0.2 * mm)
    m_ref[0:1, :] = jnp.full((1, 16), m, jnp.float32)


def _tc2(acc, b1, W2, a_src2, a_dst2):
    return pl.pallas_call(
        _tc2_body,
        out_shape=(_f32((NP, D2E)), _f32((1, NP)), _f32((1, 16))),
    )(acc, b1, W2, a_src2, a_dst2)


def _tc3_body(acc_ref, b_ref, out_ref):
    acc = acc_ref[0] + acc_ref[1]
    v = acc[:N, :D2] / acc[:N, D2:D2 + 1] + b_ref[...]
    v = v - jnp.max(v, axis=1, keepdims=True)
    out_ref[...] = v - jnp.log(jnp.sum(jnp.exp(v), axis=1, keepdims=True))


def _tc3(acc, b2):
    return pl.pallas_call(
        _tc3_body,
        out_shape=_f32((N, D2)),
    )(acc, b2)


def _sc_compiler_params():
    cp = pltpu.CompilerParams()
    fields = pltpu.CompilerParams.__dataclass_fields__
    if "needs_layout_passes" in fields:
        cp = dataclasses.replace(cp, needs_layout_passes=False)
    if "use_tc_tiling_on_sc" in fields:
        cp = dataclasses.replace(cp, use_tc_tiling_on_sc=False)
    return cp


def _sc_edges(h_pad, adst, m16, src_pad, dst_pad, zer, d, de, h_in_spmem):
    mesh = plsc.VectorSubcoreMesh(core_axis_name="c", subcore_axis_name="s")
    h_sh_shape = (NP, de) if h_in_spmem else (8, de)

    @functools.partial(
        pl.kernel,
        out_type=_f32((NC, NP, de)),
        mesh=mesh,
        compiler_params=_sc_compiler_params(),
        scratch_types=[
            pltpu.VMEM_SHARED((NP, de), jnp.float32),
            pltpu.VMEM_SHARED(h_sh_shape, jnp.float32),
            pltpu.VMEM((CPT + 2, CH), jnp.int32),
            pltpu.VMEM((CPT, CH), jnp.int32),
            pltpu.VMEM((CH, de), jnp.float32),
            pltpu.VMEM((CH, de), jnp.float32),
            pltpu.VMEM((NP,), jnp.float32),
            pltpu.VMEM((16,), jnp.float32),
            pltpu.SemaphoreType.DMA,
            pltpu.SemaphoreType.DMA,
        ],
    )
    def k(h_hbm, adst_hbm, m_hbm, src_hbm, dst_hbm, zer_hbm, acc_hbm,
          acc_sh, h_sh, src_all, dst_all, rows0, rows1,
          adst_v, m_v, sem0, sem1):
        cid = lax.axis_index("c")
        sid = lax.axis_index("s")
        wid = cid * NS + sid
        pltpu.sync_copy(adst_hbm, adst_v)
        pltpu.sync_copy(m_hbm, m_v)
        pltpu.sync_copy(src_hbm.at[wid], src_all)
        pltpu.sync_copy(dst_hbm.at[wid], dst_all)
        row0 = sid * ROWS_PER_TEC
        if h_in_spmem:
            pltpu.sync_copy(h_hbm.at[pl.ds(row0, ROWS_PER_TEC)],
                            h_sh.at[pl.ds(row0, ROWS_PER_TEC)])
        h_tab = h_sh if h_in_spmem else h_hbm
        pltpu.sync_copy(zer_hbm.at[pl.ds(row0, ROWS_PER_TEC)],
                        acc_sh.at[pl.ds(row0, ROWS_PER_TEC)])
        m_reg = m_v[...]
        plsc.subcore_barrier()

        pltpu.async_copy(h_tab.at[src_all.at[0]], rows0, sem0)
        pltpu.async_copy(h_tab.at[src_all.at[1]], rows1, sem1)

        @pl.loop(0, CPT, step=2)
        def _(ci0):
            for b, (rows_v, sem) in enumerate(((rows0, sem0), (rows1, sem1))):
                ci = ci0 + b
                pltpu.make_async_copy(h_tab.at[src_all.at[ci]], rows_v,
                                      sem).wait()

                @pl.loop(0, CH, step=16)
                def _(g):
                    dst16 = dst_all[ci, pl.ds(g, 16)]
                    row16 = lax.iota(jnp.int32, 16) + g
                    s = plsc.load_gather(
                        rows_v, [row16, jnp.full((16,), d + 1, jnp.int32)])
                    dd = plsc.load_gather(adst_v, [dst16])
                    e = s + dd
                    e = jnp.maximum(e, 0.2 * e)
                    w16 = jnp.exp(e - m_reg)
                    for c in range(d + 1):
                        col16 = jnp.full((16,), c, jnp.int32)
                        v = plsc.load_gather(rows_v, [row16, col16])
                        plsc.store_scatter(rows_v, [row16, col16], v * w16)

                pltpu.sync_copy(rows_v, acc_sh.at[dst_all.at[ci]], add=True)
                pltpu.async_copy(h_tab.at[src_all.at[ci + 2]], rows_v, sem)

        pltpu.make_async_copy(h_tab.at[src_all.at[CPT]], rows0, sem0).wait()
        pltpu.make_async_copy(h_tab.at[src_all.at[CPT + 1]], rows1,
                              sem1).wait()
        plsc.subcore_barrier()
        pltpu.sync_copy(acc_sh.at[pl.ds(row0, ROWS_PER_TEC)],
                        acc_hbm.at[cid].at[pl.ds(row0, ROWS_PER_TEC)])

    return k(h_pad, adst, m16, src_pad, dst_pad, zer)


def kernel(x, edges_index, W1, a_src1, a_dst1, b1, W2, a_src2, a_dst2, b2):
    loop = jnp.arange(N, dtype=edges_index.dtype)
    pad = jnp.full((E_PAD - E - N,), N, dtype=edges_index.dtype)
    tail = jnp.full((NW, 2, CH), N, dtype=edges_index.dtype)
    src = jnp.concatenate([edges_index[0], loop, pad]).reshape(NW, CPT, CH)
    src = jnp.concatenate([src, tail], axis=1)
    dst = jnp.concatenate([edges_index[1], loop, pad]).reshape(NW, CPT, CH)

    zer1 = jnp.zeros((NP, D1E), jnp.float32)
    zer2 = jnp.zeros((NP, D2E), jnp.float32)

    h1, adst1, m1 = _tc1(x, W1, a_src1, a_dst1)
    acc1 = _sc_edges(h1, adst1.reshape(NP), m1.reshape(16),
                     src, dst, zer1, D1, D1E, False)
    h2, adst2, m2 = _tc2(acc1, b1, W2, a_src2, a_dst2)
    acc2 = _sc_edges(h2, adst2.reshape(NP), m2.reshape(16),
                     src, dst, zer2, D2, D2E, True)
    return _tc3(acc2, b2)


# speedup vs baseline: 29.1992x; 1.5954x over previous
import dataclasses
import functools

import jax
import jax.numpy as jnp
from jax import lax
from jax.experimental import pallas as pl
from jax.experimental.pallas import tpu as pltpu
from jax.experimental.pallas import tpu_sc as plsc

N = 10000
NP = 10240
E = 320000
NC = 2
NS = 16
NW = NC * NS
CH = 112
CPT = 96
E_PAD = NW * CH * CPT
PACK_SHIFT = 14
ROWS_PER_TEC = NP // NS
NEG = -1e30
D1 = 64
D1E = 72
D2 = 40
D2E = 48


def _f32(shape):
    return jax.ShapeDtypeStruct(shape, jnp.float32)


def _fill_row(h_ref, h, asrc_col, d, de):
    h_ref[:N, :d] = h
    h_ref[N:, :d] = jnp.zeros((NP - N, d), jnp.float32)
    h_ref[:, d:d + 1] = jnp.ones((NP, 1), jnp.float32)
    h_ref[:N, d + 1:d + 2] = asrc_col
    h_ref[N:, d + 1:d + 2] = jnp.full((NP - N, 1), NEG, jnp.float32)
    h_ref[:, d + 2:] = jnp.zeros((NP, de - d - 2), jnp.float32)


def _tc1_body(x_ref, w_ref, as_ref, ad_ref, h_ref, adst_ref, m_ref):
    h = jnp.dot(x_ref[...], w_ref[...], preferred_element_type=jnp.float32)
    asrc_col = jnp.dot(h, as_ref[...][:, None],
                       preferred_element_type=jnp.float32)
    _fill_row(h_ref, h, asrc_col, D1, D1E)
    adst = jnp.sum(h * ad_ref[...], axis=1)
    adst_ref[0:1, :N] = adst[None, :]
    adst_ref[0:1, N:] = jnp.full((1, NP - N), NEG, jnp.float32)
    mm = jnp.max(asrc_col) + jnp.max(adst)
    m = jnp.maximum(mm, 0.2 * mm)
    m_ref[0:1, :] = jnp.full((1, 16), m, jnp.float32)


def _tc1(x, W1, a_src1, a_dst1):
    return pl.pallas_call(
        _tc1_body,
        out_shape=(_f32((NP, D1E)), _f32((1, NP)), _f32((1, 16))),
    )(x, W1, a_src1, a_dst1)


def _tc2_body(acc_ref, b_ref, w_ref, as_ref, ad_ref,
              h_ref, adst_ref, m_ref):
    acc = acc_ref[0] + acc_ref[1]
    h1 = acc[:N, :D1] / acc[:N, D1:D1 + 1] + b_ref[...]
    h1 = jnp.maximum(h1, 0.0)
    h2 = jnp.dot(h1, w_ref[...], preferred_element_type=jnp.float32)
    asrc_col = jnp.dot(h2, as_ref[...][:, None],
                       preferred_element_type=jnp.float32)
    _fill_row(h_ref, h2, asrc_col, D2, D2E)
    adst = jnp.sum(h2 * ad_ref[...], axis=1)
    adst_ref[0:1, :N] = adst[None, :]
    adst_ref[0:1, N:] = jnp.full((1, NP - N), NEG, jnp.float32)
    mm = jnp.max(asrc_col) + jnp.max(adst)
    m = jnp.maximum(mm, 0.2 * mm)
    m_ref[0:1, :] = jnp.full((1, 16), m, jnp.float32)


def _tc2(acc, b1, W2, a_src2, a_dst2):
    return pl.pallas_call(
        _tc2_body,
        out_shape=(_f32((NP, D2E)), _f32((1, NP)), _f32((1, 16))),
    )(acc, b1, W2, a_src2, a_dst2)


def _tc3_body(acc_ref, b_ref, out_ref):
    acc = acc_ref[0] + acc_ref[1]
    v = acc[:N, :D2] / acc[:N, D2:D2 + 1] + b_ref[...]
    v = v - jnp.max(v, axis=1, keepdims=True)
    out_ref[...] = v - jnp.log(jnp.sum(jnp.exp(v), axis=1, keepdims=True))


def _tc3(acc, b2):
    return pl.pallas_call(
        _tc3_body,
        out_shape=_f32((N, D2)),
    )(acc, b2)


def _sc_compiler_params():
    cp = pltpu.CompilerParams()
    fields = pltpu.CompilerParams.__dataclass_fields__
    if "needs_layout_passes" in fields:
        cp = dataclasses.replace(cp, needs_layout_passes=False)
    if "use_tc_tiling_on_sc" in fields:
        cp = dataclasses.replace(cp, use_tc_tiling_on_sc=False)
    return cp


def _sc_edges(h_pad, adst, m16, edges_packed, zer, d, de):
    mesh = plsc.VectorSubcoreMesh(core_axis_name="c", subcore_axis_name="s")

    @functools.partial(
        pl.kernel,
        out_type=_f32((NC, NP, de)),
        mesh=mesh,
        compiler_params=_sc_compiler_params(),
        scratch_types=[
            pltpu.VMEM_SHARED((NP, de), jnp.float32),
            pltpu.VMEM_SHARED((NP, de), jnp.float32),
            pltpu.VMEM((CPT + 2, CH), jnp.int32),
            pltpu.VMEM((CH,), jnp.int32),
            pltpu.VMEM((CH,), jnp.int32),
            pltpu.VMEM((CH,), jnp.int32),
            pltpu.VMEM((CH, de), jnp.float32),
            pltpu.VMEM((CH, de), jnp.float32),
            pltpu.VMEM((NP,), jnp.float32),
            pltpu.VMEM((16,), jnp.float32),
            pltpu.SemaphoreType.DMA,
            pltpu.SemaphoreType.DMA,
        ],
    )
    def k(h_hbm, adst_hbm, m_hbm, ep_hbm, zer_hbm, acc_hbm,
          acc_sh, h_sh, ep_all, srcb0, srcb1, dstb, rows0, rows1,
          adst_v, m_v, sem0, sem1):
        cid = lax.axis_index("c")
        sid = lax.axis_index("s")
        wid = cid * NS + sid
        pltpu.sync_copy(adst_hbm, adst_v)
        pltpu.sync_copy(m_hbm, m_v)
        pltpu.sync_copy(ep_hbm.at[wid], ep_all)
        row0 = sid * ROWS_PER_TEC
        pltpu.sync_copy(h_hbm.at[pl.ds(row0, ROWS_PER_TEC)],
                        h_sh.at[pl.ds(row0, ROWS_PER_TEC)])
        pltpu.sync_copy(zer_hbm.at[pl.ds(row0, ROWS_PER_TEC)],
                        acc_sh.at[pl.ds(row0, ROWS_PER_TEC)])
        m_reg = m_v[...]
        mask14 = jnp.full((16,), (1 << PACK_SHIFT) - 1, jnp.int32)

        def unpack_src(ci, srcb):
            @pl.loop(0, CH, step=16)
            def _(g):
                p = ep_all[ci, pl.ds(g, 16)]
                srcb[pl.ds(g, 16)] = jnp.bitwise_and(p, mask14)

        plsc.subcore_barrier()

        unpack_src(0, srcb0)
        pltpu.async_copy(h_sh.at[srcb0], rows0, sem0)
        unpack_src(1, srcb1)
        pltpu.async_copy(h_sh.at[srcb1], rows1, sem1)

        @pl.loop(0, CPT, step=2)
        def _(ci0):
            for b, (rows_v, srcb, sem) in enumerate(
                    ((rows0, srcb0, sem0), (rows1, srcb1, sem1))):
                ci = ci0 + b
                pltpu.make_async_copy(h_sh.at[srcb], rows_v, sem).wait()

                @pl.loop(0, CH, step=16)
                def _(g):
                    p = ep_all[ci, pl.ds(g, 16)]
                    dst16 = lax.shift_right_logical(p, PACK_SHIFT)
                    dstb[pl.ds(g, 16)] = dst16
                    row16 = lax.iota(jnp.int32, 16) + g
                    s = plsc.load_gather(
                        rows_v, [row16, jnp.full((16,), d + 1, jnp.int32)])
                    dd = plsc.load_gather(adst_v, [dst16])
                    e = s + dd
                    e = jnp.maximum(e, 0.2 * e)
                    w16 = jnp.exp(e - m_reg)
                    for c in range(d + 1):
                        col16 = jnp.full((16,), c, jnp.int32)
                        v = plsc.load_gather(rows_v, [row16, col16])
                        plsc.store_scatter(rows_v, [row16, col16], v * w16)

                pltpu.sync_copy(rows_v, acc_sh.at[dstb], add=True)
                unpack_src(ci + 2, srcb)
                pltpu.async_copy(h_sh.at[srcb], rows_v, sem)

        pltpu.make_async_copy(h_sh.at[srcb0], rows0, sem0).wait()
        pltpu.make_async_copy(h_sh.at[srcb1], rows1, sem1).wait()
        plsc.subcore_barrier()
        pltpu.sync_copy(acc_sh.at[pl.ds(row0, ROWS_PER_TEC)],
                        acc_hbm.at[cid].at[pl.ds(row0, ROWS_PER_TEC)])

    return k(h_pad, adst, m16, edges_packed, zer)


def kernel(x, edges_index, W1, a_src1, a_dst1, b1, W2, a_src2, a_dst2, b2):
    loop = jnp.arange(N, dtype=edges_index.dtype)
    pad = jnp.full((E_PAD - E - N,), N, dtype=edges_index.dtype)
    src = jnp.concatenate([edges_index[0], loop, pad])
    dst = jnp.concatenate([edges_index[1], loop, pad])
    sent = N | (N << PACK_SHIFT)
    packed = (src | (dst << PACK_SHIFT)).reshape(NW, CPT, CH)
    tail = jnp.full((NW, 2, CH), sent, dtype=packed.dtype)
    packed = jnp.concatenate([packed, tail], axis=1)

    zer1 = jnp.zeros((NP, D1E), jnp.float32)
    zer2 = jnp.zeros((NP, D2E), jnp.float32)

    h1, adst1, m1 = _tc1(x, W1, a_src1, a_dst1)
    acc1 = _sc_edges(h1, adst1.reshape(NP), m1.reshape(16),
                     packed, zer1, D1, D1E)
    h2, adst2, m2 = _tc2(acc1, b1, W2, a_src2, a_dst2)
    acc2 = _sc_edges(h2, adst2.reshape(NP), m2.reshape(16),
                     packed, zer2, D2, D2E)
    return _tc3(acc2, b2)


# speedup vs baseline: 32.2379x; 1.1041x over previous
import dataclasses
import functools

import jax
import jax.numpy as jnp
from jax import lax
from jax.experimental import pallas as pl
from jax.experimental.pallas import tpu as pltpu
from jax.experimental.pallas import tpu_sc as plsc

N = 10000
NP = 10112
E = 320000
NC = 2
NS = 16
NW = NC * NS
CH = 80
CPT = 131
E_PAD = NW * CH * CPT
PACK_SHIFT = 14
ROWS_PER_TEC = NP // NS
NEG = -1e30
D1 = 64
D1E = 72
D2 = 40
D2E = 48


def _f32(shape):
    return jax.ShapeDtypeStruct(shape, jnp.float32)


def _fill_row(h_ref, h, asrc_col, d, de):
    h_ref[:N, :d] = h
    h_ref[N:, :d] = jnp.zeros((NP - N, d), jnp.float32)
    h_ref[:, d:d + 1] = jnp.ones((NP, 1), jnp.float32)
    h_ref[:N, d + 1:d + 2] = asrc_col
    h_ref[N:, d + 1:d + 2] = jnp.full((NP - N, 1), NEG, jnp.float32)
    h_ref[:, d + 2:] = jnp.zeros((NP, de - d - 2), jnp.float32)


def _tc1_body(x_ref, w_ref, as_ref, ad_ref, h_ref, adst_ref, m_ref):
    h = jnp.dot(x_ref[...], w_ref[...], preferred_element_type=jnp.float32)
    asrc_col = jnp.dot(h, as_ref[...][:, None],
                       preferred_element_type=jnp.float32)
    _fill_row(h_ref, h, asrc_col, D1, D1E)
    adst = jnp.sum(h * ad_ref[...], axis=1)
    adst_ref[0:1, :N] = adst[None, :]
    adst_ref[0:1, N:] = jnp.full((1, NP - N), NEG, jnp.float32)
    mm = jnp.max(asrc_col) + jnp.max(adst)
    m = jnp.maximum(mm, 0.2 * mm)
    m_ref[0:1, :] = jnp.full((1, 16), m, jnp.float32)


def _tc1(x, W1, a_src1, a_dst1):
    return pl.pallas_call(
        _tc1_body,
        out_shape=(_f32((NP, D1E)), _f32((1, NP)), _f32((1, 16))),
    )(x, W1, a_src1, a_dst1)


def _tc2_body(acc_ref, b_ref, w_ref, as_ref, ad_ref,
              h_ref, adst_ref, m_ref):
    acc = acc_ref[0] + acc_ref[1]
    h1 = acc[:N, :D1] / acc[:N, D1:D1 + 1] + b_ref[...]
    h1 = jnp.maximum(h1, 0.0)
    h2 = jnp.dot(h1, w_ref[...], preferred_element_type=jnp.float32)
    asrc_col = jnp.dot(h2, as_ref[...][:, None],
                       preferred_element_type=jnp.float32)
    _fill_row(h_ref, h2, asrc_col, D2, D2E)
    adst = jnp.sum(h2 * ad_ref[...], axis=1)
    adst_ref[0:1, :N] = adst[None, :]
    adst_ref[0:1, N:] = jnp.full((1, NP - N), NEG, jnp.float32)
    mm = jnp.max(asrc_col) + jnp.max(adst)
    m = jnp.maximum(mm, 0.2 * mm)
    m_ref[0:1, :] = jnp.full((1, 16), m, jnp.float32)


def _tc2(acc, b1, W2, a_src2, a_dst2):
    return pl.pallas_call(
        _tc2_body,
        out_shape=(_f32((NP, D2E)), _f32((1, NP)), _f32((1, 16))),
    )(acc, b1, W2, a_src2, a_dst2)


def _tc3_body(acc_ref, b_ref, out_ref):
    acc = acc_ref[0] + acc_ref[1]
    v = acc[:N, :D2] / acc[:N, D2:D2 + 1] + b_ref[...]
    v = v - jnp.max(v, axis=1, keepdims=True)
    out_ref[...] = v - jnp.log(jnp.sum(jnp.exp(v), axis=1, keepdims=True))


def _tc3(acc, b2):
    return pl.pallas_call(
        _tc3_body,
        out_shape=_f32((N, D2)),
    )(acc, b2)


def _sc_compiler_params():
    cp = pltpu.CompilerParams()
    fields = pltpu.CompilerParams.__dataclass_fields__
    if "needs_layout_passes" in fields:
        cp = dataclasses.replace(cp, needs_layout_passes=False)
    if "use_tc_tiling_on_sc" in fields:
        cp = dataclasses.replace(cp, use_tc_tiling_on_sc=False)
    return cp


def _sc_edges(h_pad, adst, m16, edges_packed, zer, d, de):
    mesh = plsc.VectorSubcoreMesh(core_axis_name="c", subcore_axis_name="s")

    @functools.partial(
        pl.kernel,
        out_type=_f32((NC, NP, de)),
        mesh=mesh,
        compiler_params=_sc_compiler_params(),
        scratch_types=[
            pltpu.VMEM_SHARED((NP, de), jnp.float32),
            pltpu.VMEM_SHARED((NP, de), jnp.float32),
            pltpu.VMEM((CPT + 2, CH), jnp.int32),
            pltpu.VMEM((CH,), jnp.int32),
            pltpu.VMEM((CH,), jnp.int32),
            pltpu.VMEM((CH,), jnp.int32),
            pltpu.VMEM((CH,), jnp.int32),
            pltpu.VMEM((CH,), jnp.int32),
            pltpu.VMEM((CH,), jnp.int32),
            pltpu.VMEM((CH, de), jnp.float32),
            pltpu.VMEM((CH, de), jnp.float32),
            pltpu.VMEM((CH, de), jnp.float32),
            pltpu.VMEM((NP,), jnp.float32),
            pltpu.VMEM((16,), jnp.float32),
            pltpu.SemaphoreType.DMA,
            pltpu.SemaphoreType.DMA,
            pltpu.SemaphoreType.DMA,
            pltpu.SemaphoreType.DMA,
            pltpu.SemaphoreType.DMA,
            pltpu.SemaphoreType.DMA,
        ],
    )
    def k(h_hbm, adst_hbm, m_hbm, ep_hbm, zer_hbm, acc_hbm,
          acc_sh, h_sh, ep_all, srcb0, srcb1, srcb2, dstb0, dstb1, dstb2,
          rows0, rows1, rows2,
          adst_v, m_v, gsem0, gsem1, gsem2, ssem0, ssem1, ssem2):
        cid = lax.axis_index("c")
        sid = lax.axis_index("s")
        wid = cid * NS + sid
        pltpu.sync_copy(adst_hbm, adst_v)
        pltpu.sync_copy(m_hbm, m_v)
        pltpu.sync_copy(ep_hbm.at[wid], ep_all)
        row0 = sid * ROWS_PER_TEC
        pltpu.sync_copy(h_hbm.at[pl.ds(row0, ROWS_PER_TEC)],
                        h_sh.at[pl.ds(row0, ROWS_PER_TEC)])
        pltpu.sync_copy(zer_hbm.at[pl.ds(row0, ROWS_PER_TEC)],
                        acc_sh.at[pl.ds(row0, ROWS_PER_TEC)])
        m_reg = m_v[...]
        mask14 = jnp.full((16,), (1 << PACK_SHIFT) - 1, jnp.int32)
        rows = (rows0, rows1, rows2)
        srcb = (srcb0, srcb1, srcb2)
        dstb = (dstb0, dstb1, dstb2)
        gsem = (gsem0, gsem1, gsem2)
        ssem = (ssem0, ssem1, ssem2)

        def unpack_src(ci, b):
            @pl.loop(0, CH, step=16)
            def _(g):
                p = ep_all[ci, pl.ds(g, 16)]
                srcb[b][pl.ds(g, 16)] = jnp.bitwise_and(p, mask14)

        def issue_gather(ci, b):
            unpack_src(ci, b)
            pltpu.async_copy(h_sh.at[srcb[b]], rows[b], gsem[b])

        def compute(ci, b):
            pltpu.make_async_copy(h_sh.at[srcb[b]], rows[b],
                                  gsem[b]).wait()
            rows_v = rows[b]

            @pl.loop(0, CH, step=16)
            def _(g):
                p = ep_all[ci, pl.ds(g, 16)]
                dst16 = lax.shift_right_logical(p, PACK_SHIFT)
                dstb[b][pl.ds(g, 16)] = dst16
                row16 = lax.iota(jnp.int32, 16) + g
                s = plsc.load_gather(
                    rows_v, [row16, jnp.full((16,), d + 1, jnp.int32)])
                dd = plsc.load_gather(adst_v, [dst16])
                e = s + dd
                e = jnp.maximum(e, 0.2 * e)
                w16 = jnp.exp(e - m_reg)
                for c in range(d + 1):
                    col16 = jnp.full((16,), c, jnp.int32)
                    v = plsc.load_gather(rows_v, [row16, col16])
                    plsc.store_scatter(rows_v, [row16, col16], v * w16)

            pltpu.async_copy(rows_v, acc_sh.at[dstb[b]], ssem[b],
                             add=True)

        def wait_scatter(b):
            pltpu.make_async_copy(rows[b], acc_sh.at[dstb[b]],
                                  ssem[b]).wait()

        plsc.subcore_barrier()

        issue_gather(0, 0)
        issue_gather(1, 1)
        compute(0, 0)
        issue_gather(2, 2)
        compute(1, 1)
        wait_scatter(0)
        issue_gather(3, 0)

        @pl.loop(2, CPT, step=3)
        def _(ci0):
            for k in range(3):
                ci = ci0 + k
                b = (2 + k) % 3
                compute(ci, b)
                wait_scatter((b + 2) % 3)
                issue_gather(ci + 2, (b + 2) % 3)

        wait_scatter((CPT - 1) % 3)
        pltpu.make_async_copy(h_sh.at[srcb[CPT % 3]], rows[CPT % 3],
                              gsem[CPT % 3]).wait()
        pltpu.make_async_copy(h_sh.at[srcb[(CPT + 1) % 3]],
                              rows[(CPT + 1) % 3],
                              gsem[(CPT + 1) % 3]).wait()
        plsc.subcore_barrier()
        pltpu.sync_copy(acc_sh.at[pl.ds(row0, ROWS_PER_TEC)],
                        acc_hbm.at[cid].at[pl.ds(row0, ROWS_PER_TEC)])

    return k(h_pad, adst, m16, edges_packed, zer)


def kernel(x, edges_index, W1, a_src1, a_dst1, b1, W2, a_src2, a_dst2, b2):
    loop = jnp.arange(N, dtype=edges_index.dtype)
    pad = jnp.full((E_PAD - E - N,), N, dtype=edges_index.dtype)
    src = jnp.concatenate([edges_index[0], loop, pad])
    dst = jnp.concatenate([edges_index[1], loop, pad])
    sent = N | (N << PACK_SHIFT)
    packed = (src | (dst << PACK_SHIFT)).reshape(NW, CPT, CH)
    tail = jnp.full((NW, 2, CH), sent, dtype=packed.dtype)
    packed = jnp.concatenate([packed, tail], axis=1)

    zer1 = jnp.zeros((NP, D1E), jnp.float32)
    zer2 = jnp.zeros((NP, D2E), jnp.float32)

    h1, adst1, m1 = _tc1(x, W1, a_src1, a_dst1)
    acc1 = _sc_edges(h1, adst1.reshape(NP), m1.reshape(16),
                     packed, zer1, D1, D1E)
    h2, adst2, m2 = _tc2(acc1, b1, W2, a_src2, a_dst2)
    acc2 = _sc_edges(h2, adst2.reshape(NP), m2.reshape(16),
                     packed, zer2, D2, D2E)
    return _tc3(acc2, b2)
